# baseline (speedup 1.0000x reference)
import functools

import jax
import jax.numpy as jnp
from jax import lax
from jax.experimental import pallas as pl
from jax.experimental.pallas import tpu as pltpu
from jax.experimental.pallas import tpu_sc as plsc

SEQ, B, D, H, N, K = 32, 32, 1024, 128, 16, 2
NSLOT = N + 1
L = 16
HC = H // L

COL_G = 0
COL_Q = COL_G + N
COL_K = COL_Q + H
COL_V = COL_K + H * NSLOT
NOUT_RAW = COL_V + H * NSLOT
NOUT = 4608
BLK_N = 512


def _gemm_body(x_ref, w_ref, b_ref, o_ref):
    o_ref[...] = (
        jnp.dot(x_ref[...], w_ref[...], preferred_element_type=jnp.float32)
        + b_ref[...]
    )


def _tc_gemm(x, w, b):
    m = x.shape[0]
    return pl.pallas_call(
        _gemm_body,
        grid=(NOUT // BLK_N,),
        in_specs=[
            pl.BlockSpec((m, D), lambda j: (0, 0)),
            pl.BlockSpec((D, BLK_N), lambda j: (0, j)),
            pl.BlockSpec((1, BLK_N), lambda j: (0, j)),
        ],
        out_specs=pl.BlockSpec((m, BLK_N), lambda j: (0, j)),
        out_shape=jax.ShapeDtypeStruct((m, NOUT), jnp.float32),
    )(x, w, b)


def _scalar(v):
    return v if getattr(v, "ndim", 0) == 0 else v[0]


def _make_sc_kernel():
    info = plsc.get_sparse_core_info()
    nc = info.num_cores
    mesh = plsc.VectorSubcoreMesh(core_axis_name="c", subcore_axis_name="s")

    @functools.partial(
        pl.kernel,
        mesh=mesh,
        compiler_params=pltpu.CompilerParams(needs_layout_passes=False),
        out_type=[
            jax.ShapeDtypeStruct((SEQ * B * H,), jnp.float32),
            jax.ShapeDtypeStruct((B * NSLOT, H, H), jnp.float32),
        ],
        scratch_types=[
            pltpu.VMEM((H, H), jnp.float32),
            pltpu.VMEM((H, H), jnp.float32),
            pltpu.VMEM((H, H), jnp.float32),
            pltpu.VMEM((L,), jnp.float32),
            pltpu.VMEM((H,), jnp.float32),
            pltpu.VMEM((H,), jnp.float32),
            pltpu.VMEM((H,), jnp.float32),
            pltpu.VMEM((H,), jnp.float32),
            pltpu.VMEM((H,), jnp.float32),
            pltpu.VMEM((H,), jnp.float32),
            pltpu.VMEM((H,), jnp.float32),
            pltpu.VMEM((H,), jnp.float32),
            pltpu.SemaphoreType.DMA,
            pltpu.SemaphoreType.DMA,
            pltpu.SemaphoreType.DMA,
            pltpu.SemaphoreType.DMA,
        ],
    )
    def sc_fn(y_hbm, m0_hbm, o_hbm, m_hbm,
              blkA, blkB, blkC, lg, qv, kA, vA, kB, vB, kC, vC, ov,
              semB, semC, semW, semP):
        b = lax.axis_index("s") * nc + lax.axis_index("c")

        pltpu.sync_copy(m0_hbm.at[b * NSLOT], blkA)
        pltpu.sync_copy(
            m0_hbm.at[pl.ds(b * NSLOT + 1, NSLOT - 1)],
            m_hbm.at[pl.ds(b * NSLOT + 1, NSLOT - 1)],
        )

        row0 = b
        pltpu.async_copy(y_hbm.at[pl.ds(row0 * NOUT + COL_G, L)], lg, semP)
        pltpu.async_copy(y_hbm.at[pl.ds(row0 * NOUT + COL_Q, H)], qv, semP)
        pltpu.async_copy(y_hbm.at[pl.ds(row0 * NOUT + COL_K, H)], kA, semP)
        pltpu.async_copy(y_hbm.at[pl.ds(row0 * NOUT + COL_V, H)], vA, semP)

        def rank1_and_dot(blk, kv, vv, upd_w, acc_scale):
            vvcs = [vv[pl.ds(c * L, L)] for c in range(HC)]

            def rcloop(rc, accs):
                accs = list(accs)
                base = rc * L
                k16 = kv[pl.ds(base, L)] * upd_w
                q16 = qv[pl.ds(base, L)]
                for rl in range(L):
                    kr = k16[rl]
                    qr = q16[rl]
                    r = base + rl
                    for c in range(HC):
                        sl = pl.ds(c * L, L)
                        mrow = blk[r, sl] + kr * vvcs[c]
                        blk[r, sl] = mrow
                        accs[c] = accs[c] + qr * mrow
                return tuple(accs)

            accs = lax.fori_loop(
                0, H // L, rcloop,
                tuple(jnp.zeros((L,), jnp.float32) for _ in range(HC)),
            )
            for c in range(HC):
                sl = pl.ds(c * L, L)
                ov[sl] = ov[sl] + acc_scale * accs[c]

        def step(t, carry):
            row = t * B + b
            pltpu.make_async_copy(y_hbm.at[pl.ds(row * NOUT + COL_G, L)], lg, semP).wait()
            pltpu.make_async_copy(y_hbm.at[pl.ds(row * NOUT + COL_Q, H)], qv, semP).wait()
            pltpu.make_async_copy(y_hbm.at[pl.ds(row * NOUT + COL_K, H)], kA, semP).wait()
            pltpu.make_async_copy(y_hbm.at[pl.ds(row * NOUT + COL_V, H)], vA, semP).wait()
            l = lg[...]
            iot = lax.iota(jnp.int32, 16)
            skeys, svals = plsc.sort_key_val(l, iot, descending=True)
            idx0 = svals[0]
            idx1 = svals[1]
            b0 = jnp.full((L,), skeys[0], dtype=jnp.float32)
            b1 = jnp.full((L,), skeys[1], dtype=jnp.float32)
            ev = jnp.exp(b1 - b0)
            one = jnp.full((L,), 1.0, dtype=jnp.float32)
            g0 = one / (one + ev)
            g1 = ev * g0

            i0z = jnp.where(jnp.full((L,), idx0) == 0, 1.0, 0.0)
            i1z = jnp.where(jnp.full((L,), idx1) == 0, 1.0, 0.0)
            c0 = one + i0z + i1z
            wA = one + g0 * i0z + g1 * i1z

            @pl.when(idx0 != 0)
            def _():
                pltpu.async_copy(m_hbm.at[b * NSLOT + idx0], blkB, semB)
                pltpu.async_copy(y_hbm.at[pl.ds(row * NOUT + COL_K + idx0 * H, H)], kB, semB)
                pltpu.async_copy(y_hbm.at[pl.ds(row * NOUT + COL_V + idx0 * H, H)], vB, semB)

            @pl.when(idx1 != 0)
            def _():
                pltpu.async_copy(m_hbm.at[b * NSLOT + idx1], blkC, semC)
                pltpu.async_copy(y_hbm.at[pl.ds(row * NOUT + COL_K + idx1 * H, H)], kC, semC)
                pltpu.async_copy(y_hbm.at[pl.ds(row * NOUT + COL_V + idx1 * H, H)], vC, semC)

            for c in range(HC):
                ov[pl.ds(c * L, L)] = jnp.zeros((L,), jnp.float32)

            rank1_and_dot(blkA, kA, vA, c0, wA)

            @pl.when(idx0 != 0)
            def _():
                pltpu.make_async_copy(m_hbm.at[b * NSLOT + idx0], blkB, semB).wait()
                pltpu.make_async_copy(y_hbm.at[pl.ds(0, H)], kB, semB).wait()
                pltpu.make_async_copy(y_hbm.at[pl.ds(0, H)], vB, semB).wait()
                rank1_and_dot(blkB, kB, vB, one, g0)
                pltpu.async_copy(blkB, m_hbm.at[b * NSLOT + idx0], semW)

            @pl.when(idx1 != 0)
            def _():
                pltpu.make_async_copy(m_hbm.at[b * NSLOT + idx1], blkC, semC).wait()
                pltpu.make_async_copy(y_hbm.at[pl.ds(0, H)], kC, semC).wait()
                pltpu.make_async_copy(y_hbm.at[pl.ds(0, H)], vC, semC).wait()
                rank1_and_dot(blkC, kC, vC, one, g1)
                pltpu.async_copy(blkC, m_hbm.at[b * NSLOT + idx1], semW)

            pltpu.async_copy(ov, o_hbm.at[pl.ds(row * H, H)], semW)

            nrow = jnp.minimum(t + 1, SEQ - 1) * B + b
            pltpu.async_copy(y_hbm.at[pl.ds(nrow * NOUT + COL_G, L)], lg, semP)
            pltpu.async_copy(y_hbm.at[pl.ds(nrow * NOUT + COL_Q, H)], qv, semP)
            pltpu.async_copy(y_hbm.at[pl.ds(nrow * NOUT + COL_K, H)], kA, semP)
            pltpu.async_copy(y_hbm.at[pl.ds(nrow * NOUT + COL_V, H)], vA, semP)

            @pl.when(idx0 != 0)
            def _():
                pltpu.make_async_copy(blkB, m_hbm.at[b * NSLOT + idx0], semW).wait()

            @pl.when(idx1 != 0)
            def _():
                pltpu.make_async_copy(blkC, m_hbm.at[b * NSLOT + idx1], semW).wait()

            pltpu.make_async_copy(ov, o_hbm.at[pl.ds(row * H, H)], semW).wait()
            return carry

        lax.fori_loop(0, SEQ, step, 0)

        lrow = (SEQ - 1) * B + b
        pltpu.make_async_copy(y_hbm.at[pl.ds(lrow * NOUT + COL_G, L)], lg, semP).wait()
        pltpu.make_async_copy(y_hbm.at[pl.ds(lrow * NOUT + COL_Q, H)], qv, semP).wait()
        pltpu.make_async_copy(y_hbm.at[pl.ds(lrow * NOUT + COL_K, H)], kA, semP).wait()
        pltpu.make_async_copy(y_hbm.at[pl.ds(lrow * NOUT + COL_V, H)], vA, semP).wait()
        pltpu.sync_copy(blkA, m_hbm.at[b * NSLOT])

    return sc_fn


def kernel(X, M_0, Wk, bk, Wv, bv, Wg, bg, Wq, bq):
    x_flat = X.reshape(SEQ * B, D)
    pad = jnp.zeros((D, NOUT - NOUT_RAW), jnp.float32)
    w_cat = jnp.concatenate([Wg, Wq, Wk, Wv, pad], axis=1)
    b_cat = jnp.concatenate(
        [bg, bq, bk, bv, jnp.zeros((NOUT - NOUT_RAW,), jnp.float32)]
    ).reshape(1, NOUT)
    y = _tc_gemm(x_flat, w_cat, b_cat)
    o_flat, m_flat = _make_sc_kernel()(y.reshape(-1), M_0.reshape(B * NSLOT, H, H))
    return o_flat.reshape(SEQ, B, H), m_flat.reshape(B, NSLOT, H, H)


# speedup vs baseline: 5.0160x; 5.0160x over previous
import functools

import jax
import jax.numpy as jnp
from jax import lax
from jax.experimental import pallas as pl
from jax.experimental.pallas import tpu as pltpu
from jax.experimental.pallas import tpu_sc as plsc

SEQ, B, D, H, N, K = 32, 32, 1024, 128, 16, 2
NSLOT = N + 1
L = 16
HC = H // L
RB = 64
GQR = 8
KVR = 24


def _gemm_body(x_ref, wgq_ref, wk_ref, wv_ref, bgq_ref, bk_ref, bv_ref,
               ogq_ref, ok_ref, ov_ref):
    xb = x_ref[...]
    ogq_ref[:, 0:2, :] = (
        jnp.dot(xb, wgq_ref[...], preferred_element_type=jnp.float32)
        + bgq_ref[...]
    ).reshape(RB, 2, H)
    ok_ref[:, 0:NSLOT, :] = (
        jnp.dot(xb, wk_ref[...], preferred_element_type=jnp.float32)
        + bk_ref[...]
    ).reshape(RB, NSLOT, H)
    ov_ref[:, 0:NSLOT, :] = (
        jnp.dot(xb, wv_ref[...], preferred_element_type=jnp.float32)
        + bv_ref[...]
    ).reshape(RB, NSLOT, H)


def _tc_gemm(x, wgq, wk, wv, bgq, bk, bv):
    m = x.shape[0]
    grid = (m // RB,)
    return pl.pallas_call(
        _gemm_body,
        grid=grid,
        in_specs=[
            pl.BlockSpec((RB, D), lambda j: (j, 0)),
            pl.BlockSpec((D, 2 * H), lambda j: (0, 0)),
            pl.BlockSpec((D, NSLOT * H), lambda j: (0, 0)),
            pl.BlockSpec((D, NSLOT * H), lambda j: (0, 0)),
            pl.BlockSpec((1, 2 * H), lambda j: (0, 0)),
            pl.BlockSpec((1, NSLOT * H), lambda j: (0, 0)),
            pl.BlockSpec((1, NSLOT * H), lambda j: (0, 0)),
        ],
        out_specs=[
            pl.BlockSpec((RB, GQR, H), lambda j: (j, 0, 0)),
            pl.BlockSpec((RB, KVR, H), lambda j: (j, 0, 0)),
            pl.BlockSpec((RB, KVR, H), lambda j: (j, 0, 0)),
        ],
        out_shape=[
            jax.ShapeDtypeStruct((m, GQR, H), jnp.float32),
            jax.ShapeDtypeStruct((m, KVR, H), jnp.float32),
            jax.ShapeDtypeStruct((m, KVR, H), jnp.float32),
        ],
    )(x, wgq, wk, wv, bgq, bk, bv)


def _make_sc_kernel():
    info = plsc.get_sparse_core_info()
    nc = info.num_cores
    mesh = plsc.VectorSubcoreMesh(core_axis_name="c", subcore_axis_name="s")

    @functools.partial(
        pl.kernel,
        mesh=mesh,
        compiler_params=pltpu.CompilerParams(needs_layout_passes=False),
        out_type=[
            jax.ShapeDtypeStruct((SEQ * B * H,), jnp.float32),
            jax.ShapeDtypeStruct((B * NSLOT, H, H), jnp.float32),
        ],
        scratch_types=[
            pltpu.VMEM((H, H), jnp.float32),
            pltpu.VMEM((H, H), jnp.float32),
            pltpu.VMEM((H, H), jnp.float32),
            pltpu.VMEM((2, GQR, H), jnp.float32),
            pltpu.VMEM((2, KVR, H), jnp.float32),
            pltpu.VMEM((2, KVR, H), jnp.float32),
            pltpu.VMEM((H,), jnp.float32),
            pltpu.SemaphoreType.DMA,
            pltpu.SemaphoreType.DMA,
            pltpu.SemaphoreType.DMA,
            pltpu.SemaphoreType.DMA,
        ],
    )
    def sc_fn(gq_hbm, k_hbm, v_hbm, m0_hbm, o_hbm, m_hbm,
              blkA, blkB, blkC, gqr, krr, vrr, ov,
              semB, semC, semW, semP):
        b = lax.axis_index("s") * nc + lax.axis_index("c")

        pltpu.sync_copy(m0_hbm.at[b * NSLOT], blkA)
        ibufs = (blkB, blkC)
        isems = (semB, semC)
        pltpu.async_copy(m0_hbm.at[b * NSLOT + 1], blkB, semB)
        for s in range(1, NSLOT):
            buf = ibufs[(s - 1) % 2]
            sem = isems[(s - 1) % 2]
            pltpu.make_async_copy(m0_hbm.at[b * NSLOT + s], buf, sem).wait()
            pltpu.async_copy(buf, m_hbm.at[b * NSLOT + s], sem)
            if s + 1 < NSLOT:
                nbuf = ibufs[s % 2]
                nsem = isems[s % 2]
                if s >= 2:
                    pltpu.make_async_copy(
                        nbuf, m_hbm.at[b * NSLOT + s - 1], nsem
                    ).wait()
                pltpu.async_copy(m0_hbm.at[b * NSLOT + s + 1], nbuf, nsem)
        pltpu.make_async_copy(blkB, m_hbm.at[b * NSLOT], semB).wait()
        pltpu.make_async_copy(blkC, m_hbm.at[b * NSLOT], semC).wait()

        pltpu.async_copy(gq_hbm.at[b], gqr.at[0], semP)
        pltpu.async_copy(k_hbm.at[b], krr.at[0], semP)
        pltpu.async_copy(v_hbm.at[b], vrr.at[0], semP)

        def rank1_and_dot(blk, par, slot, upd_w, acc_scale, rc_lo, rc_hi):
            vvcs = [vrr[par, slot, pl.ds(c * L, L)] for c in range(HC)]

            def rcloop(rc, accs):
                accs = list(accs)
                base = rc * L
                k16 = krr[par, slot, pl.ds(base, L)] * upd_w
                q16 = gqr[par, 1, pl.ds(base, L)]
                for rl in range(L):
                    kr = k16[rl]
                    qr = q16[rl]
                    r = base + rl
                    for c in range(HC):
                        sl = pl.ds(c * L, L)
                        mrow = blk[r, sl] + kr * vvcs[c]
                        blk[r, sl] = mrow
                        accs[c] = accs[c] + qr * mrow
                return tuple(accs)

            accs = lax.fori_loop(
                rc_lo, rc_hi, rcloop,
                tuple(jnp.zeros((L,), jnp.float32) for _ in range(HC)),
            )
            for c in range(HC):
                sl = pl.ds(c * L, L)
                ov[sl] = ov[sl] + acc_scale * accs[c]

        def step(t, carry):
            row = t * B + b
            par = t % 2
            nxt = 1 - par
            pltpu.make_async_copy(gq_hbm.at[row], gqr.at[par], semP).wait()
            pltpu.make_async_copy(k_hbm.at[row], krr.at[par], semP).wait()
            pltpu.make_async_copy(v_hbm.at[row], vrr.at[par], semP).wait()
            nrow = jnp.minimum(t + 1, SEQ - 1) * B + b
            pltpu.async_copy(gq_hbm.at[nrow], gqr.at[nxt], semP)
            pltpu.async_copy(k_hbm.at[nrow], krr.at[nxt], semP)
            pltpu.async_copy(v_hbm.at[nrow], vrr.at[nxt], semP)

            l = gqr[par, 0, pl.ds(0, L)]
            iot = lax.iota(jnp.int32, 16)
            skeys, svals = plsc.sort_key_val(l, iot, descending=True)
            idx0 = svals[0]
            idx1 = svals[1]
            b0 = jnp.full((L,), skeys[0], dtype=jnp.float32)
            b1 = jnp.full((L,), skeys[1], dtype=jnp.float32)
            ev = jnp.exp(b1 - b0)
            one = jnp.full((L,), 1.0, dtype=jnp.float32)
            g0 = one / (one + ev)
            g1 = ev * g0
            i0z = jnp.where(jnp.full((L,), idx0) == 0, 1.0, 0.0)
            i1z = jnp.where(jnp.full((L,), idx1) == 0, 1.0, 0.0)
            c0 = one + i0z + i1z
            wA = one + g0 * i0z + g1 * i1z

            @pl.when(idx0 != 0)
            def _():
                pltpu.async_copy(m_hbm.at[b * NSLOT + idx0], blkB, semB)

            @pl.when(idx1 != 0)
            def _():
                pltpu.async_copy(m_hbm.at[b * NSLOT + idx1], blkC, semC)

            for c in range(HC):
                ov[pl.ds(c * L, L)] = jnp.zeros((L,), jnp.float32)

            rank1_and_dot(blkA, par, 0, c0, wA, 0, HC // 2)

            @pl.when(idx0 != 0)
            def _():
                pltpu.make_async_copy(m_hbm.at[b * NSLOT + idx0], blkB, semB).wait()
                rank1_and_dot(blkB, par, idx0, one, g0, 0, HC)
                pltpu.async_copy(blkB, m_hbm.at[b * NSLOT + idx0], semW)

            @pl.when(idx1 != 0)
            def _():
                pltpu.make_async_copy(m_hbm.at[b * NSLOT + idx1], blkC, semC).wait()
                rank1_and_dot(blkC, par, idx1, one, g1, 0, HC)
                pltpu.async_copy(blkC, m_hbm.at[b * NSLOT + idx1], semW)

            rank1_and_dot(blkA, par, 0, c0, wA, HC // 2, HC)

            pltpu.async_copy(ov, o_hbm.at[pl.ds(row * H, H)], semW)

            @pl.when(idx0 != 0)
            def _():
                pltpu.make_async_copy(blkB, m_hbm.at[b * NSLOT + idx0], semW).wait()

            @pl.when(idx1 != 0)
            def _():
                pltpu.make_async_copy(blkC, m_hbm.at[b * NSLOT + idx1], semW).wait()

            pltpu.make_async_copy(ov, o_hbm.at[pl.ds(row * H, H)], semW).wait()
            return carry

        lax.fori_loop(0, SEQ, step, 0)

        lrow = (SEQ - 1) * B + b
        pltpu.make_async_copy(gq_hbm.at[lrow], gqr.at[0], semP).wait()
        pltpu.make_async_copy(k_hbm.at[lrow], krr.at[0], semP).wait()
        pltpu.make_async_copy(v_hbm.at[lrow], vrr.at[0], semP).wait()
        pltpu.sync_copy(blkA, m_hbm.at[b * NSLOT])

    return sc_fn


def kernel(X, M_0, Wk, bk, Wv, bv, Wg, bg, Wq, bq):
    x_flat = X.reshape(SEQ * B, D)
    zpad = jnp.zeros((D, H - N), jnp.float32)
    wgq = jnp.concatenate([Wg, zpad, Wq], axis=1)
    bgq = jnp.concatenate(
        [bg, jnp.zeros((H - N,), jnp.float32), bq]
    ).reshape(1, 2 * H)
    ygq, yk, yv = _tc_gemm(
        x_flat, wgq, Wk, Wv, bgq, bk.reshape(1, -1), bv.reshape(1, -1)
    )
    o_flat, m_flat = _make_sc_kernel()(
        ygq, yk, yv, M_0.reshape(B * NSLOT, H, H)
    )
    return o_flat.reshape(SEQ, B, H), m_flat.reshape(B, NSLOT, H, H)
